# baseline (device time: 99359 ns/iter reference)
import jax
import jax.numpy as jnp
from jax import lax
from jax.experimental import pallas as pl
from jax.experimental.pallas import tpu as pltpu

N_DEV = 8


def kernel(A, B):
    m, k = A.shape
    k2, n = B.shape

    def body(a_ref, b_ref, out_ref, comm_ref, send_sems, recv_sems):
        my = lax.axis_index("i")
        left = (my - 1) % N_DEV
        right = (my + 1) % N_DEV

        barrier_sem = pltpu.get_barrier_semaphore()
        for nbr in [left, right]:
            pl.semaphore_signal(
                barrier_sem, inc=1,
                device_id=(nbr,), device_id_type=pl.DeviceIdType.MESH,
            )
        pl.semaphore_wait(barrier_sem, 2)

        partial = jnp.dot(a_ref[:, :], b_ref[:, :],
                          preferred_element_type=jnp.float32)
        comm_ref[0, :, :] = partial
        out_ref[:, :] = partial

        for h in range(N_DEV - 1):
            rdma = pltpu.make_async_remote_copy(
                src_ref=comm_ref.at[h],
                dst_ref=comm_ref.at[h + 1],
                send_sem=send_sems.at[h],
                recv_sem=recv_sems.at[h + 1],
                device_id=(right,),
                device_id_type=pl.DeviceIdType.MESH,
            )
            rdma.start()
            rdma.wait()
            out_ref[:, :] += comm_ref[h + 1, :, :]

        z = out_ref[:, :]
        out_ref[:, :] = z / (1.0 + jnp.exp(-z))

    return pl.pallas_call(
        body,
        out_shape=jax.ShapeDtypeStruct((m, n), jnp.float32),
        in_specs=[
            pl.BlockSpec(memory_space=pltpu.VMEM),
            pl.BlockSpec(memory_space=pltpu.VMEM),
        ],
        out_specs=pl.BlockSpec(memory_space=pltpu.VMEM),
        scratch_shapes=[
            pltpu.VMEM((N_DEV, m, n), jnp.float32),
            pltpu.SemaphoreType.DMA((N_DEV,)),
            pltpu.SemaphoreType.DMA((N_DEV,)),
        ],
        compiler_params=pltpu.CompilerParams(collective_id=0),
    )(A, B)


# device time: 24438 ns/iter; 4.0658x vs baseline; 4.0658x over previous
import jax
import jax.numpy as jnp
from jax import lax
from jax.experimental import pallas as pl
from jax.experimental.pallas import tpu as pltpu

N_DEV = 8


def kernel(A, B):
    m, k = A.shape
    k2, n = B.shape
    mc = m // N_DEV

    def body(a_ref, b_ref, out_ref, part_ref, p1_ref,
             send_p1, recv_p1, send_p2, recv_p2):
        my = lax.axis_index("i")

        barrier_sem = pltpu.get_barrier_semaphore()
        for d in range(1, N_DEV):
            pl.semaphore_signal(
                barrier_sem, inc=1,
                device_id=((my + d) % N_DEV,),
                device_id_type=pl.DeviceIdType.MESH,
            )
        pl.semaphore_wait(barrier_sem, N_DEV - 1)

        part_ref[:, :, :] = jnp.dot(
            a_ref[:, :], b_ref[:, :], preferred_element_type=jnp.float32
        ).reshape(N_DEV, mc, n)

        p1_ref[pl.ds(my, 1), :, :] = part_ref[pl.ds(my, 1), :, :]

        p1_sends = []
        for d in range(1, N_DEV):
            tgt = (my + d) % N_DEV
            rdma = pltpu.make_async_remote_copy(
                src_ref=part_ref.at[tgt],
                dst_ref=p1_ref.at[my],
                send_sem=send_p1.at[d - 1],
                recv_sem=recv_p1.at[my],
                device_id=(tgt,),
                device_id_type=pl.DeviceIdType.MESH,
            )
            rdma.start()
            p1_sends.append(rdma)

        for d in range(1, N_DEV):
            src = (my + d) % N_DEV
            recv = pltpu.make_async_remote_copy(
                src_ref=p1_ref.at[src],
                dst_ref=p1_ref.at[src],
                send_sem=send_p1.at[d - 1],
                recv_sem=recv_p1.at[src],
                device_id=(src,),
                device_id_type=pl.DeviceIdType.MESH,
            )
            recv.wait_recv()

        z = jnp.sum(p1_ref[:, :, :], axis=0)
        out_ref[pl.ds(my * mc, mc), :] = z / (1.0 + jnp.exp(-z))

        p2_sends = []
        for d in range(1, N_DEV):
            tgt = (my + d) % N_DEV
            rdma = pltpu.make_async_remote_copy(
                src_ref=out_ref.at[pl.ds(my * mc, mc), :],
                dst_ref=out_ref.at[pl.ds(my * mc, mc), :],
                send_sem=send_p2.at[d - 1],
                recv_sem=recv_p2.at[my],
                device_id=(tgt,),
                device_id_type=pl.DeviceIdType.MESH,
            )
            rdma.start()
            p2_sends.append(rdma)

        for rdma in p1_sends:
            rdma.wait_send()

        for d in range(1, N_DEV):
            src = (my + d) % N_DEV
            recv = pltpu.make_async_remote_copy(
                src_ref=out_ref.at[pl.ds(src * mc, mc), :],
                dst_ref=out_ref.at[pl.ds(src * mc, mc), :],
                send_sem=send_p2.at[d - 1],
                recv_sem=recv_p2.at[src],
                device_id=(src,),
                device_id_type=pl.DeviceIdType.MESH,
            )
            recv.wait_recv()

        for rdma in p2_sends:
            rdma.wait_send()

    return pl.pallas_call(
        body,
        out_shape=jax.ShapeDtypeStruct((m, n), jnp.float32),
        in_specs=[
            pl.BlockSpec(memory_space=pltpu.VMEM),
            pl.BlockSpec(memory_space=pltpu.VMEM),
        ],
        out_specs=pl.BlockSpec(memory_space=pltpu.VMEM),
        scratch_shapes=[
            pltpu.VMEM((N_DEV, m // N_DEV, n), jnp.float32),
            pltpu.VMEM((N_DEV, m // N_DEV, n), jnp.float32),
            pltpu.SemaphoreType.DMA((N_DEV - 1,)),
            pltpu.SemaphoreType.DMA((N_DEV,)),
            pltpu.SemaphoreType.DMA((N_DEV - 1,)),
            pltpu.SemaphoreType.DMA((N_DEV,)),
        ],
        compiler_params=pltpu.CompilerParams(collective_id=0),
    )(A, B)


# device time: 18475 ns/iter; 5.3780x vs baseline; 1.3228x over previous
import jax
import jax.numpy as jnp
from jax import lax
from jax.experimental import pallas as pl
from jax.experimental.pallas import tpu as pltpu

N_DEV = 8


def kernel(A, B):
    m, k = A.shape
    k2, n = B.shape
    mc = m // N_DEV

    def body(a_ref, b_ref, out_ref, part_ref, p1_ref, g_ref,
             send_p1, recv_p1, send_p2, recv_p2):
        my = lax.axis_index("i")

        barrier_sem = pltpu.get_barrier_semaphore()
        for d in range(1, N_DEV):
            pl.semaphore_signal(
                barrier_sem, inc=1,
                device_id=((my + d) % N_DEV,),
                device_id_type=pl.DeviceIdType.MESH,
            )

        part_ref[:, :, :] = (
            jnp.dot(a_ref[:, :], b_ref[:, :],
                    preferred_element_type=jnp.float32)
            .reshape(N_DEV, mc, n)
            .astype(jnp.bfloat16)
        )
        p1_ref[pl.ds(my, 1), :, :] = part_ref[pl.ds(my, 1), :, :]

        pl.semaphore_wait(barrier_sem, N_DEV - 1)

        p1_sends = []
        for d in range(1, N_DEV):
            tgt = (my + d) % N_DEV
            rdma = pltpu.make_async_remote_copy(
                src_ref=part_ref.at[tgt],
                dst_ref=p1_ref.at[my],
                send_sem=send_p1.at[d - 1],
                recv_sem=recv_p1.at[my],
                device_id=(tgt,),
                device_id_type=pl.DeviceIdType.MESH,
            )
            rdma.start()
            p1_sends.append(rdma)

        for d in range(1, N_DEV):
            src = (my + d) % N_DEV
            recv = pltpu.make_async_remote_copy(
                src_ref=p1_ref.at[src],
                dst_ref=p1_ref.at[src],
                send_sem=send_p1.at[d - 1],
                recv_sem=recv_p1.at[src],
                device_id=(src,),
                device_id_type=pl.DeviceIdType.MESH,
            )
            recv.wait_recv()

        z = jnp.sum(p1_ref[:, :, :].astype(jnp.float32), axis=0)
        g_ref[pl.ds(my, 1), :, :] = (
            (z / (1.0 + jnp.exp(-z))).astype(jnp.bfloat16)[None]
        )

        p2_sends = []
        for d in range(1, N_DEV):
            tgt = (my + d) % N_DEV
            rdma = pltpu.make_async_remote_copy(
                src_ref=g_ref.at[my],
                dst_ref=g_ref.at[my],
                send_sem=send_p2.at[d - 1],
                recv_sem=recv_p2.at[my],
                device_id=(tgt,),
                device_id_type=pl.DeviceIdType.MESH,
            )
            rdma.start()
            p2_sends.append(rdma)

        for rdma in p1_sends:
            rdma.wait_send()

        for d in range(1, N_DEV):
            src = (my + d) % N_DEV
            recv = pltpu.make_async_remote_copy(
                src_ref=g_ref.at[src],
                dst_ref=g_ref.at[src],
                send_sem=send_p2.at[d - 1],
                recv_sem=recv_p2.at[src],
                device_id=(src,),
                device_id_type=pl.DeviceIdType.MESH,
            )
            recv.wait_recv()

        out_ref[:, :] = g_ref[:, :, :].astype(jnp.float32).reshape(m, n)

        for rdma in p2_sends:
            rdma.wait_send()

    return pl.pallas_call(
        body,
        out_shape=jax.ShapeDtypeStruct((m, n), jnp.float32),
        in_specs=[
            pl.BlockSpec(memory_space=pltpu.VMEM),
            pl.BlockSpec(memory_space=pltpu.VMEM),
        ],
        out_specs=pl.BlockSpec(memory_space=pltpu.VMEM),
        scratch_shapes=[
            pltpu.VMEM((N_DEV, m // N_DEV, n), jnp.bfloat16),
            pltpu.VMEM((N_DEV, m // N_DEV, n), jnp.bfloat16),
            pltpu.VMEM((N_DEV, m // N_DEV, n), jnp.bfloat16),
            pltpu.SemaphoreType.DMA((N_DEV - 1,)),
            pltpu.SemaphoreType.DMA((N_DEV,)),
            pltpu.SemaphoreType.DMA((N_DEV - 1,)),
            pltpu.SemaphoreType.DMA((N_DEV,)),
        ],
        compiler_params=pltpu.CompilerParams(collective_id=0),
    )(A, B)


# device time: 12570 ns/iter; 7.9045x vs baseline; 1.4698x over previous
import jax
import jax.numpy as jnp
from jax import lax
from jax.experimental import pallas as pl
from jax.experimental.pallas import tpu as pltpu

N_DEV = 8


def kernel(A, B):
    m, k = A.shape
    k2, n = B.shape
    mc = m // N_DEV

    def body(a_ref, b_ref, out_ref, part_ref, p1_ref, g_ref,
             send_p1, recv_p1, send_p2, recv_p2):
        my = lax.axis_index("i")

        barrier_sem = pltpu.get_barrier_semaphore()
        for d in range(1, N_DEV):
            pl.semaphore_signal(
                barrier_sem, inc=1,
                device_id=((my + d) % N_DEV,),
                device_id_type=pl.DeviceIdType.MESH,
            )

        part_ref[:, :, :] = jnp.dot(
            a_ref[:, :].astype(jnp.bfloat16),
            b_ref[:, :].astype(jnp.bfloat16),
            preferred_element_type=jnp.float32,
        ).astype(jnp.bfloat16).reshape(N_DEV, mc, n)
        p1_ref[pl.ds(my, 1), :, :] = part_ref[pl.ds(my, 1), :, :]

        pl.semaphore_wait(barrier_sem, N_DEV - 1)

        p1_sends = []
        for d in range(1, N_DEV):
            tgt = (my + d) % N_DEV
            rdma = pltpu.make_async_remote_copy(
                src_ref=part_ref.at[tgt],
                dst_ref=p1_ref.at[my],
                send_sem=send_p1.at[d - 1],
                recv_sem=recv_p1.at[my],
                device_id=(tgt,),
                device_id_type=pl.DeviceIdType.MESH,
            )
            rdma.start()
            p1_sends.append(rdma)

        z = p1_ref[pl.ds(my, 1), :, :].astype(jnp.float32)
        for d in range(1, N_DEV):
            src = (my + d) % N_DEV
            recv = pltpu.make_async_remote_copy(
                src_ref=p1_ref.at[src],
                dst_ref=p1_ref.at[src],
                send_sem=send_p1.at[d - 1],
                recv_sem=recv_p1.at[src],
                device_id=(src,),
                device_id_type=pl.DeviceIdType.MESH,
            )
            recv.wait_recv()
            z += p1_ref[pl.ds(src, 1), :, :].astype(jnp.float32)

        z = z[0]
        silu = z / (1.0 + jnp.exp(-z))
        g_ref[pl.ds(my, 1), :, :] = silu.astype(jnp.bfloat16)[None]

        p2_sends = []
        for d in range(1, N_DEV):
            tgt = (my + d) % N_DEV
            rdma = pltpu.make_async_remote_copy(
                src_ref=g_ref.at[my],
                dst_ref=g_ref.at[my],
                send_sem=send_p2.at[d - 1],
                recv_sem=recv_p2.at[my],
                device_id=(tgt,),
                device_id_type=pl.DeviceIdType.MESH,
            )
            rdma.start()
            p2_sends.append(rdma)

        out_ref[pl.ds(my * mc, mc), :] = silu

        for rdma in p1_sends:
            rdma.wait_send()

        for d in range(1, N_DEV):
            src = (my + d) % N_DEV
            recv = pltpu.make_async_remote_copy(
                src_ref=g_ref.at[src],
                dst_ref=g_ref.at[src],
                send_sem=send_p2.at[d - 1],
                recv_sem=recv_p2.at[src],
                device_id=(src,),
                device_id_type=pl.DeviceIdType.MESH,
            )
            recv.wait_recv()
            out_ref[pl.ds(src * mc, mc), :] = (
                g_ref[pl.ds(src, 1), :, :].astype(jnp.float32)[0]
            )

        for rdma in p2_sends:
            rdma.wait_send()

    return pl.pallas_call(
        body,
        out_shape=jax.ShapeDtypeStruct((m, n), jnp.float32),
        in_specs=[
            pl.BlockSpec(memory_space=pltpu.VMEM),
            pl.BlockSpec(memory_space=pltpu.VMEM),
        ],
        out_specs=pl.BlockSpec(memory_space=pltpu.VMEM),
        scratch_shapes=[
            pltpu.VMEM((N_DEV, m // N_DEV, n), jnp.bfloat16),
            pltpu.VMEM((N_DEV, m // N_DEV, n), jnp.bfloat16),
            pltpu.VMEM((N_DEV, m // N_DEV, n), jnp.bfloat16),
            pltpu.SemaphoreType.DMA((N_DEV - 1,)),
            pltpu.SemaphoreType.DMA((N_DEV,)),
            pltpu.SemaphoreType.DMA((N_DEV - 1,)),
            pltpu.SemaphoreType.DMA((N_DEV,)),
        ],
        compiler_params=pltpu.CompilerParams(collective_id=0),
    )(A, B)
